# baseline (speedup 1.0000x reference)
import functools

import jax
import jax.numpy as jnp
from jax import lax
from jax.experimental import pallas as pl
from jax.experimental.pallas import tpu as pltpu
from jax.experimental.pallas import tpu_sc as plsc

LOGZERO = -1.0e10
B, T, O = 8, 256, 1000
H = 4
NBH = B * H
W = 256
L = 16
NCHUNK = W // L
WH = W // 2
LN2 = 0.6931471805599453


def _vlog(s):
    i = lax.bitcast_convert_type(s, jnp.int32)
    e = lax.shift_right_arithmetic(i, 23) - 127
    m = lax.bitcast_convert_type(
        jnp.bitwise_or(jnp.bitwise_and(i, jnp.int32(0x7FFFFF)),
                       jnp.int32(127 << 23)), jnp.float32)
    z = (m - 1.0) / (m + 1.0)
    z2 = z * z
    p = 2.0 * z * (1.0 + z2 * (1.0 / 3.0 + z2 * (1.0 / 5.0
                   + z2 * (1.0 / 7.0 + z2 * (1.0 / 9.0)))))
    return e.astype(jnp.float32) * LN2 + p


def _splat_i32(v):
    return jnp.full((L,), v, dtype=jnp.int32)


_mesh = plsc.VectorSubcoreMesh(core_axis_name="c", subcore_axis_name="s")


@functools.partial(
    pl.kernel, mesh=_mesh,
    out_type=jax.ShapeDtypeStruct((NBH * O,), jnp.float32),
    compiler_params=pltpu.CompilerParams(needs_layout_passes=False),
    scratch_types=[
        pltpu.VMEM((W, T), jnp.float32),
        pltpu.VMEM((8, T), jnp.float32),
        pltpu.VMEM((T,), jnp.float32),
        pltpu.VMEM((B,), jnp.int32),
        pltpu.VMEM((W,), jnp.float32),
        pltpu.SemaphoreType.DMA,
        pltpu.SemaphoreType.DMA,
        pltpu.SemaphoreType.DMA,
    ],
)
def _ctc_kernel(x_hbm, xlens_hbm, out_hbm, xv, xb, cv, xlv, ov,
                semb, sem1, sem2):
    wid = lax.axis_index("c") * 16 + lax.axis_index("s")
    b = wid // 4
    k = wid % 4
    o0 = k * W - (k // 3) * 24

    cpb = pltpu.async_copy(x_hbm.at[b, pl.ds(0, 8), :], xb, semb)
    cp1 = pltpu.async_copy(x_hbm.at[b, pl.ds(o0, WH), :],
                           xv.at[pl.ds(0, WH)], sem1)
    cp2 = pltpu.async_copy(x_hbm.at[b, pl.ds(o0 + WH, WH), :],
                           xv.at[pl.ds(WH, WH)], sem2)
    pltpu.sync_copy(xlens_hbm, xlv)

    lanes = lax.iota(jnp.int32, L)
    xlen = jnp.max(plsc.load_gather(xlv, [_splat_i32(b)]))
    xlen_v = _splat_i32(0) + xlen

    cpb.wait()
    lzv = jnp.full((L,), LOGZERO, dtype=jnp.float32)
    carry = jnp.float32(0.0)
    csv = []
    for kc in range(T // L):
        blk = xb[0, pl.ds(kc * L, L)]
        inc = plsc.cumsum(blk) + carry
        cv[pl.ds(kc * L, L)] = inc
        exc = inc - blk
        exc = jnp.where((lanes + (kc * L)) >= xlen_v, lzv, exc)
        if kc == 0:
            exc = jnp.where(lanes == 0, lzv, exc)
        csv.append(exc)
        carry = carry + jnp.sum(blk)

    zgate = jnp.where(xlen == T, jnp.float32(0.0), jnp.float32(LOGZERO))
    cend = plsc.load_gather(cv, [_splat_i32(xlen - 1)])
    fzero = jnp.zeros((L,), dtype=jnp.float32)
    minit = jnp.full((L,), LOGZERO, dtype=jnp.float32)
    rows = [lanes + (j * L) for j in range(NCHUNK)]

    def obody(u, carry, j=None):
        macc, sacc = carry
        o = j * L + u
        vs = [csv[tc] + xv[o, pl.ds(tc * L, L)] for tc in range(T // L)]
        t8 = [jnp.maximum(vs[i], vs[i + 8]) for i in range(8)]
        t4 = [jnp.maximum(t8[i], t8[i + 4]) for i in range(4)]
        t2 = [jnp.maximum(t4[i], t4[i + 2]) for i in range(2)]
        ms = jnp.max(jnp.maximum(t2[0], t2[1]))
        es = [jnp.exp(v - ms) for v in vs]
        e8 = [es[i] + es[i + 8] for i in range(8)]
        e4 = [e8[i] + e8[i + 4] for i in range(4)]
        e2 = [e4[i] + e4[i + 2] for i in range(2)]
        ss = jnp.sum(e2[0] + e2[1])
        macc = jnp.where(lanes == u, ms, macc)
        sacc = jnp.where(lanes == u, ss, sacc)
        return macc, sacc

    cp1.wait()
    mv, sv = [], []
    for j in range(NCHUNK):
        if j == NCHUNK // 2:
            cp2.wait()
        macc, sacc = lax.fori_loop(0, L, functools.partial(obody, j=j),
                                   (minit, fzero))
        mv.append(macc)
        sv.append(sacc)

    G3 = 4
    ptot = []
    for g in range(NCHUNK // G3):
        def sfx_loop(g=g):
            def sfx_body(i, carry, g=g):
                t = T - 1 - i
                ts = _splat_i32(t)
                cvec = plsc.load_gather(cv, [_splat_i32(t - 1)])
                out = []
                for u in range(G3):
                    s, p = carry[2 * u], carry[2 * u + 1]
                    p = p + plsc.load_gather(xv, [rows[g * G3 + u], ts])
                    s = s + jnp.exp(cvec + p - mv[g * G3 + u])
                    out.extend((s, p))
                return tuple(out)
            return lax.fori_loop(0, T - 1, sfx_body, (fzero,) * (2 * G3))

        def sfx_skip():
            return (fzero,) * (2 * G3)

        res = lax.cond(xlen == T, sfx_loop, sfx_skip)
        for u in range(G3):
            sv[g * G3 + u] = sv[g * G3 + u] + res[2 * u]
            ptot.append(res[2 * u + 1])

    tz = _splat_i32(0)
    for j in range(NCHUNK):
        a0 = plsc.load_gather(xv, [rows[j], tz])
        term0 = ptot[j] + a0 + zgate
        m2 = jnp.maximum(mv[j], term0)
        s2 = sv[j] * jnp.exp(mv[j] - m2) + jnp.exp(term0 - m2)
        res = m2 + _vlog(s2)
        oabs = lanes + (o0 + j * L)
        res = jnp.where(oabs == (O - 1), cend, res)
        res = jnp.where(oabs == 0, jnp.float32(LOGZERO), res)
        ov[pl.ds(j * L, L)] = res

    for h in range(H):
        pltpu.sync_copy(ov, out_hbm.at[pl.ds((b * H + h) * O + o0, W)])


def kernel(x, xlens, y):
    del y
    xt = jnp.transpose(x, (0, 2, 1))
    return _ctc_kernel(xt, xlens.astype(jnp.int32)).reshape(NBH, O)


# speedup vs baseline: 1.0353x; 1.0353x over previous
import functools

import jax
import jax.numpy as jnp
from jax import lax
from jax.experimental import pallas as pl
from jax.experimental.pallas import tpu as pltpu
from jax.experimental.pallas import tpu_sc as plsc

LOGZERO = -1.0e10
B, T, O = 8, 256, 1000
H = 4
NBH = B * H
W = 256
L = 16
NCHUNK = W // L
WH = W // 2
LN2 = 0.6931471805599453


def _vlog(s):
    i = lax.bitcast_convert_type(s, jnp.int32)
    e = lax.shift_right_arithmetic(i, 23) - 127
    m = lax.bitcast_convert_type(
        jnp.bitwise_or(jnp.bitwise_and(i, jnp.int32(0x7FFFFF)),
                       jnp.int32(127 << 23)), jnp.float32)
    z = (m - 1.0) / (m + 1.0)
    z2 = z * z
    p = 2.0 * z * (1.0 + z2 * (1.0 / 3.0 + z2 * (1.0 / 5.0
                   + z2 * (1.0 / 7.0 + z2 * (1.0 / 9.0)))))
    return e.astype(jnp.float32) * LN2 + p


def _splat_i32(v):
    return jnp.full((L,), v, dtype=jnp.int32)


_mesh = plsc.VectorSubcoreMesh(core_axis_name="c", subcore_axis_name="s")


@functools.partial(
    pl.kernel, mesh=_mesh,
    out_type=jax.ShapeDtypeStruct((NBH * O,), jnp.float32),
    compiler_params=pltpu.CompilerParams(needs_layout_passes=False),
    scratch_types=[
        pltpu.VMEM((W, T), jnp.float32),
        pltpu.VMEM((8, T), jnp.float32),
        pltpu.VMEM((T,), jnp.float32),
        pltpu.VMEM((B,), jnp.int32),
        pltpu.VMEM((W,), jnp.float32),
        pltpu.SemaphoreType.DMA,
        pltpu.SemaphoreType.DMA,
        pltpu.SemaphoreType.DMA,
    ],
)
def _ctc_kernel(x_hbm, xlens_hbm, out_hbm, xv, xb, cv, xlv, ov,
                semb, sem1, sem2):
    wid = lax.axis_index("c") * 16 + lax.axis_index("s")
    b = wid // 4
    k = wid % 4
    o0 = k * W - (k // 3) * 24

    cpb = pltpu.async_copy(x_hbm.at[b, pl.ds(0, 8), :], xb, semb)
    cp1 = pltpu.async_copy(x_hbm.at[b, pl.ds(o0, WH), :],
                           xv.at[pl.ds(0, WH)], sem1)
    cp2 = pltpu.async_copy(x_hbm.at[b, pl.ds(o0 + WH, WH), :],
                           xv.at[pl.ds(WH, WH)], sem2)
    pltpu.sync_copy(xlens_hbm, xlv)

    lanes = lax.iota(jnp.int32, L)
    xlen = jnp.max(plsc.load_gather(xlv, [_splat_i32(b)]))
    xlen_v = _splat_i32(0) + xlen

    cpb.wait()
    lzv = jnp.full((L,), LOGZERO, dtype=jnp.float32)
    carry = jnp.float32(0.0)
    csv = []
    for kc in range(T // L):
        blk = xb[0, pl.ds(kc * L, L)]
        inc = plsc.cumsum(blk) + carry
        cv[pl.ds(kc * L, L)] = inc
        exc = inc - blk
        exc = jnp.where((lanes + (kc * L)) >= xlen_v, lzv, exc)
        if kc == 0:
            exc = jnp.where(lanes == 0, lzv, exc)
        csv.append(exc)
        carry = carry + jnp.sum(blk)

    zgate = jnp.where(xlen == T, jnp.float32(0.0), jnp.float32(LOGZERO))
    cend = plsc.load_gather(cv, [_splat_i32(xlen - 1)])
    fzero = jnp.zeros((L,), dtype=jnp.float32)
    minit = jnp.full((L,), LOGZERO, dtype=jnp.float32)
    rows = [lanes + (j * L) for j in range(NCHUNK)]

    def obody(u, carry, j=None):
        macc, sacc = carry
        o = j * L + u
        avs = [xv[o, pl.ds(tc * L, L)] for tc in range(T // L)]
        m = minit
        for tc in range(T // L):
            m = jnp.maximum(m, csv[tc] + avs[tc])
        ms = jnp.max(m)
        s = fzero
        for tc in range(T // L):
            s = s + jnp.exp(csv[tc] + avs[tc] - ms)
        ss = jnp.sum(s)
        macc = jnp.where(lanes == u, ms, macc)
        sacc = jnp.where(lanes == u, ss, sacc)
        return macc, sacc

    cp1.wait()
    mv, sv = [], []
    for j in range(NCHUNK):
        if j == NCHUNK // 2:
            cp2.wait()
        macc, sacc = lax.fori_loop(0, L, functools.partial(obody, j=j),
                                   (minit, fzero))
        mv.append(macc)
        sv.append(sacc)

    G3 = 4
    ptot = []
    for g in range(NCHUNK // G3):
        def sfx_loop(g=g):
            def sfx_body(i, carry, g=g):
                t = T - 1 - i
                ts = _splat_i32(t)
                cvec = plsc.load_gather(cv, [_splat_i32(t - 1)])
                out = []
                for u in range(G3):
                    s, p = carry[2 * u], carry[2 * u + 1]
                    p = p + plsc.load_gather(xv, [rows[g * G3 + u], ts])
                    s = s + jnp.exp(cvec + p - mv[g * G3 + u])
                    out.extend((s, p))
                return tuple(out)
            return lax.fori_loop(0, T - 1, sfx_body, (fzero,) * (2 * G3))

        def sfx_skip():
            return (fzero,) * (2 * G3)

        res = lax.cond(xlen == T, sfx_loop, sfx_skip)
        for u in range(G3):
            sv[g * G3 + u] = sv[g * G3 + u] + res[2 * u]
            ptot.append(res[2 * u + 1])

    tz = _splat_i32(0)
    for j in range(NCHUNK):
        a0 = plsc.load_gather(xv, [rows[j], tz])
        term0 = ptot[j] + a0 + zgate
        m2 = jnp.maximum(mv[j], term0)
        s2 = sv[j] * jnp.exp(mv[j] - m2) + jnp.exp(term0 - m2)
        res = m2 + _vlog(s2)
        oabs = lanes + (o0 + j * L)
        res = jnp.where(oabs == (O - 1), cend, res)
        res = jnp.where(oabs == 0, jnp.float32(LOGZERO), res)
        ov[pl.ds(j * L, L)] = res

    for h in range(H):
        pltpu.sync_copy(ov, out_hbm.at[pl.ds((b * H + h) * O + o0, W)])


def kernel(x, xlens, y):
    del y
    xt = jnp.transpose(x, (0, 2, 1))
    return _ctc_kernel(xt, xlens.astype(jnp.int32)).reshape(NBH, O)
